# Initial kernel scaffold; baseline (speedup 1.0000x reference)
import jax
import jax.numpy as jnp
from jax.experimental import pallas as pl


def kernel(x, edge_index, W0, b0, W1, b1, W2, b2, Wl, bl):
    raise NotImplementedError("write your pallas kernel here")


# speedup vs baseline: 7.4796x; 7.4796x over previous
import functools

import jax
import jax.numpy as jnp
from jax import lax
from jax.experimental import pallas as pl
from jax.experimental.pallas import tpu as pltpu
from jax.experimental.pallas import tpu_sc as plsc

F32 = jnp.float32
NTILES = 16
CH = 128
GRP = 8


def _prescale_body(x_ref, d0_ref, d1_ref, w_ref, o0_ref, o1_ref):
    h = o0_ref.shape[1]
    dinv = lax.rsqrt(d0_ref[...] + d1_ref[...])
    xw = jnp.dot(x_ref[...], w_ref[...], preferred_element_type=F32)
    xs = xw * dinv
    o0_ref[...] = xs[:, :h]
    o1_ref[...] = xs[:, h:]


def _layer_body(a0_ref, a1_ref, d0_ref, d1_ref, b_ref, w_ref, o0_ref, o1_ref):
    h = o0_ref.shape[1]
    dinv = lax.rsqrt(d0_ref[...] + d1_ref[...])
    agg = jnp.concatenate([a0_ref[...], a1_ref[...]], axis=1)
    act = jnp.maximum(agg * dinv + b_ref[...], 0.0)
    xw = jnp.dot(act, w_ref[...], preferred_element_type=F32)
    xs = xw * dinv
    o0_ref[...] = xs[:, :h]
    o1_ref[...] = xs[:, h:]


def _final_body(a0_ref, a1_ref, d0_ref, d1_ref, b_ref, w_ref, bl_ref, o_ref):
    dinv = lax.rsqrt(d0_ref[...] + d1_ref[...])
    agg = jnp.concatenate([a0_ref[...], a1_ref[...]], axis=1)
    act = jnp.maximum(agg * dinv + b_ref[...], 0.0)
    o_ref[...] = jnp.dot(act, w_ref[...], preferred_element_type=F32) + bl_ref[...]


def _make_deg_kernel(npad, kg):
    rpt = npad // NTILES
    kg2 = kg // 2
    mesh = plsc.VectorSubcoreMesh(core_axis_name="c", subcore_axis_name="s",
                                  num_cores=2, num_subcores=NTILES)

    @functools.partial(
        pl.kernel,
        out_type=(jax.ShapeDtypeStruct((npad,), F32),
                  jax.ShapeDtypeStruct((npad,), F32)),
        mesh=mesh,
        scratch_types=[
            pltpu.VMEM_SHARED((npad,), F32),
            pltpu.VMEM((GRP, CH), jnp.int32),
            pltpu.VMEM((CH,), F32),
        ],
    )
    def deg_kernel(dst_hbm, init_hbm, out0, out1, dacc, dst_b, ones_v):
        c = lax.axis_index("c")
        s = lax.axis_index("s")
        r0 = s * rpt
        pltpu.sync_copy(init_hbm.at[pl.ds(c * npad + r0, rpt)],
                        dacc.at[pl.ds(r0, rpt)])
        pltpu.sync_copy(init_hbm.at[pl.ds(0, CH)], ones_v)
        plsc.subcore_barrier()

        @pl.loop(c * kg2, (c + 1) * kg2)
        def _(g):
            pltpu.sync_copy(dst_hbm.at[s, g], dst_b)
            for j in range(GRP):
                pltpu.sync_copy(ones_v, dacc.at[dst_b.at[j]], add=True)

        plsc.subcore_barrier()

        @pl.when(c == 0)
        def _():
            pltpu.sync_copy(dacc.at[pl.ds(r0, rpt)], out0.at[pl.ds(r0, rpt)])

        @pl.when(c == 1)
        def _():
            pltpu.sync_copy(dacc.at[pl.ds(r0, rpt)], out1.at[pl.ds(r0, rpt)])

    return deg_kernel


def _make_agg_kernel(npad, h, kg):
    rpt = npad // NTILES
    mesh = plsc.VectorSubcoreMesh(core_axis_name="c", subcore_axis_name="s",
                                  num_cores=2, num_subcores=NTILES)

    @functools.partial(
        pl.kernel,
        out_type=(jax.ShapeDtypeStruct((npad, h), F32),
                  jax.ShapeDtypeStruct((npad, h), F32)),
        mesh=mesh,
        scratch_types=[
            pltpu.VMEM_SHARED((npad, h), F32),
            pltpu.VMEM((GRP, CH), jnp.int32),
            pltpu.VMEM((GRP, CH), jnp.int32),
            pltpu.VMEM((CH, h), F32),
            pltpu.SemaphoreType.DMA,
        ],
    )
    def agg_kernel(xs0, xs1, src_hbm, dst_hbm, out0, out1,
                   acc, src_b, dst_b, rows, sem):
        c = lax.axis_index("c")
        s = lax.axis_index("s")
        r0 = s * rpt

        def run(xs_c, out_c):
            pltpu.sync_copy(xs_c.at[pl.ds(r0, rpt)], acc.at[pl.ds(r0, rpt)])
            plsc.subcore_barrier()

            @pl.loop(0, kg)
            def _(g):
                pltpu.sync_copy(src_hbm.at[s, g], src_b)
                pltpu.sync_copy(dst_hbm.at[s, g], dst_b)
                for j in range(GRP):
                    pltpu.async_copy(xs_c.at[src_b.at[j]], rows, sem).wait()
                    pltpu.sync_copy(rows, acc.at[dst_b.at[j]], add=True)

            plsc.subcore_barrier()
            pltpu.sync_copy(acc.at[pl.ds(r0, rpt)], out_c.at[pl.ds(r0, rpt)])

        @pl.when(c == 0)
        def _():
            run(xs0, out0)

        @pl.when(c == 1)
        def _():
            run(xs1, out1)

    return agg_kernel


def _row_spec(rb, w):
    return pl.BlockSpec((rb, w), lambda i: (i, 0))


def _full_spec(shape):
    nd = len(shape)
    return pl.BlockSpec(shape, lambda i: (0,) * nd)


@jax.jit
def kernel(x, edge_index, W0, b0, W1, b1, W2, b2, Wl, bl):
    n, din = x.shape
    e = edge_index.shape[1]
    dh = W0.shape[1]
    dout = Wl.shape[1]
    h = dh // 2

    rb = 1024
    npad = pl.cdiv(n + NTILES, rb) * rb
    epb = NTILES * CH * GRP * 2
    epad = pl.cdiv(e, epb) * epb
    kg = epad // (NTILES * CH * GRP)
    grid = npad // rb

    pad_e = epad - e
    src = jnp.concatenate(
        [edge_index[0],
         jnp.zeros((pad_e,), jnp.int32)]).reshape(NTILES, kg, GRP, CH)
    dst = jnp.concatenate(
        [edge_index[1],
         n + (jnp.arange(pad_e, dtype=jnp.int32) % NTILES)]
    ).reshape(NTILES, kg, GRP, CH)
    xp = jnp.zeros((npad, din), F32).at[:n].set(x)
    init = jnp.concatenate([jnp.ones((npad,), F32), jnp.zeros((npad,), F32)])

    deg0, deg1 = _make_deg_kernel(npad, kg)(dst, init)
    d0 = deg0.reshape(npad, 1)
    d1 = deg1.reshape(npad, 1)

    dspec = _row_spec(rb, 1)
    agg = _make_agg_kernel(npad, h, kg)

    xs0, xs1 = pl.pallas_call(
        _prescale_body,
        grid=(grid,),
        in_specs=[_row_spec(rb, din), dspec, dspec, _full_spec((din, dh))],
        out_specs=[_row_spec(rb, h), _row_spec(rb, h)],
        out_shape=[jax.ShapeDtypeStruct((npad, h), F32)] * 2,
    )(xp, d0, d1, W0)

    for b, w in ((b0, W1), (b1, W2)):
        a0, a1 = agg(xs0, xs1, src, dst)
        xs0, xs1 = pl.pallas_call(
            _layer_body,
            grid=(grid,),
            in_specs=[_row_spec(rb, h), _row_spec(rb, h), dspec, dspec,
                      _full_spec((1, dh)), _full_spec((dh, dh))],
            out_specs=[_row_spec(rb, h), _row_spec(rb, h)],
            out_shape=[jax.ShapeDtypeStruct((npad, h), F32)] * 2,
        )(a0, a1, d0, d1, b.reshape(1, dh), w)

    a0, a1 = agg(xs0, xs1, src, dst)
    out = pl.pallas_call(
        _final_body,
        grid=(grid,),
        in_specs=[_row_spec(rb, h), _row_spec(rb, h), dspec, dspec,
                  _full_spec((1, dh)), _full_spec((dh, dout)),
                  _full_spec((1, dout))],
        out_specs=_row_spec(rb, dout),
        out_shape=jax.ShapeDtypeStruct((npad, dout), F32),
    )(a0, a1, d0, d1, b2.reshape(1, dh), Wl, bl.reshape(1, dout))
    return out[:n]


# speedup vs baseline: 9.3175x; 1.2457x over previous
import functools

import jax
import jax.numpy as jnp
from jax import lax
from jax.experimental import pallas as pl
from jax.experimental.pallas import tpu as pltpu
from jax.experimental.pallas import tpu_sc as plsc

F32 = jnp.float32
NTILES = 16
CH = 128
GRP = 16


def _prescale_body(x_ref, d0_ref, d1_ref, w_ref, o0_ref, o1_ref):
    h = o0_ref.shape[1]
    dinv = lax.rsqrt(d0_ref[...] + d1_ref[...])
    xw = jnp.dot(x_ref[...], w_ref[...], preferred_element_type=F32)
    xs = xw * dinv
    o0_ref[...] = xs[:, :h]
    o1_ref[...] = xs[:, h:]


def _layer_body(a0_ref, a1_ref, d0_ref, d1_ref, b_ref, w_ref, o0_ref, o1_ref):
    h = o0_ref.shape[1]
    dinv = lax.rsqrt(d0_ref[...] + d1_ref[...])
    agg = jnp.concatenate([a0_ref[...], a1_ref[...]], axis=1)
    act = jnp.maximum(agg * dinv + b_ref[...], 0.0)
    xw = jnp.dot(act, w_ref[...], preferred_element_type=F32)
    xs = xw * dinv
    o0_ref[...] = xs[:, :h]
    o1_ref[...] = xs[:, h:]


def _final_body(a0_ref, a1_ref, d0_ref, d1_ref, b_ref, w_ref, bl_ref, o_ref):
    dinv = lax.rsqrt(d0_ref[...] + d1_ref[...])
    agg = jnp.concatenate([a0_ref[...], a1_ref[...]], axis=1)
    act = jnp.maximum(agg * dinv + b_ref[...], 0.0)
    o_ref[...] = jnp.dot(act, w_ref[...], preferred_element_type=F32) + bl_ref[...]


def _make_deg_kernel(npad, kg):
    rpt = npad // NTILES
    kg2 = kg // 2
    mesh = plsc.VectorSubcoreMesh(core_axis_name="c", subcore_axis_name="s",
                                  num_cores=2, num_subcores=NTILES)

    @functools.partial(
        pl.kernel,
        out_type=(jax.ShapeDtypeStruct((npad,), F32),
                  jax.ShapeDtypeStruct((npad,), F32)),
        mesh=mesh,
        scratch_types=[
            pltpu.VMEM_SHARED((npad,), F32),
            pltpu.VMEM((GRP, CH), jnp.int32),
            pltpu.VMEM((CH,), F32),
        ],
    )
    def deg_kernel(dst_hbm, init_hbm, out0, out1, dacc, dst_b, ones_v):
        c = lax.axis_index("c")
        s = lax.axis_index("s")
        r0 = s * rpt
        pltpu.sync_copy(init_hbm.at[pl.ds(c * npad + r0, rpt)],
                        dacc.at[pl.ds(r0, rpt)])
        pltpu.sync_copy(init_hbm.at[pl.ds(0, CH)], ones_v)
        plsc.subcore_barrier()

        @pl.loop(c * kg2, (c + 1) * kg2)
        def _(g):
            pltpu.sync_copy(dst_hbm.at[s, g], dst_b)
            for j in range(GRP):
                pltpu.sync_copy(ones_v, dacc.at[dst_b.at[j]], add=True)

        plsc.subcore_barrier()

        @pl.when(c == 0)
        def _():
            pltpu.sync_copy(dacc.at[pl.ds(r0, rpt)], out0.at[pl.ds(r0, rpt)])

        @pl.when(c == 1)
        def _():
            pltpu.sync_copy(dacc.at[pl.ds(r0, rpt)], out1.at[pl.ds(r0, rpt)])

    return deg_kernel


def _make_agg_kernel(npad, h, kg):
    rpt = npad // NTILES
    mesh = plsc.VectorSubcoreMesh(core_axis_name="c", subcore_axis_name="s",
                                  num_cores=2, num_subcores=NTILES)

    @functools.partial(
        pl.kernel,
        out_type=(jax.ShapeDtypeStruct((npad, h), F32),
                  jax.ShapeDtypeStruct((npad, h), F32)),
        mesh=mesh,
        scratch_types=[
            pltpu.VMEM_SHARED((npad, h), F32),
            pltpu.VMEM((GRP, CH), jnp.int32),
            pltpu.VMEM((GRP, CH), jnp.int32),
            pltpu.VMEM((CH, h), F32),
            pltpu.VMEM((CH, h), F32),
            pltpu.SemaphoreType.DMA,
            pltpu.SemaphoreType.DMA,
        ],
    )
    def agg_kernel(xs0, xs1, src_hbm, dst_hbm, out0, out1,
                   acc, src_b, dst_b, rows_a, rows_b, sem_a, sem_b):
        c = lax.axis_index("c")
        s = lax.axis_index("s")
        r0 = s * rpt
        bufs = ((rows_a, sem_a), (rows_b, sem_b))

        def run(xs_c, out_c):
            pltpu.sync_copy(xs_c.at[pl.ds(r0, rpt)], acc.at[pl.ds(r0, rpt)])
            plsc.subcore_barrier()

            @pl.loop(0, kg)
            def _(g):
                pltpu.sync_copy(src_hbm.at[s, g], src_b)
                pltpu.sync_copy(dst_hbm.at[s, g], dst_b)
                pltpu.async_copy(xs_c.at[src_b.at[0]], rows_a, sem_a)
                for j in range(GRP):
                    rows, sem = bufs[j % 2]
                    if j + 1 < GRP:
                        nrows, nsem = bufs[(j + 1) % 2]
                        pltpu.async_copy(xs_c.at[src_b.at[j + 1]], nrows, nsem)
                    pltpu.make_async_copy(xs_c.at[src_b.at[j]], rows, sem).wait()
                    pltpu.sync_copy(rows, acc.at[dst_b.at[j]], add=True)

            plsc.subcore_barrier()
            pltpu.sync_copy(acc.at[pl.ds(r0, rpt)], out_c.at[pl.ds(r0, rpt)])

        @pl.when(c == 0)
        def _():
            run(xs0, out0)

        @pl.when(c == 1)
        def _():
            run(xs1, out1)

    return agg_kernel


def _row_spec(rb, w):
    return pl.BlockSpec((rb, w), lambda i: (i, 0))


def _full_spec(shape):
    nd = len(shape)
    return pl.BlockSpec(shape, lambda i: (0,) * nd)


@jax.jit
def kernel(x, edge_index, W0, b0, W1, b1, W2, b2, Wl, bl):
    n, din = x.shape
    e = edge_index.shape[1]
    dh = W0.shape[1]
    dout = Wl.shape[1]
    h = dh // 2

    rb = 1024
    npad = pl.cdiv(n + NTILES, rb) * rb
    epb = NTILES * CH * GRP * 2
    epad = pl.cdiv(e, epb) * epb
    kg = epad // (NTILES * CH * GRP)
    grid = npad // rb

    pad_e = epad - e
    src = jnp.concatenate(
        [edge_index[0],
         jnp.zeros((pad_e,), jnp.int32)]).reshape(NTILES, kg, GRP, CH)
    dst = jnp.concatenate(
        [edge_index[1],
         n + (jnp.arange(pad_e, dtype=jnp.int32) % NTILES)]
    ).reshape(NTILES, kg, GRP, CH)
    xp = jnp.zeros((npad, din), F32).at[:n].set(x)
    init = jnp.concatenate([jnp.ones((npad,), F32), jnp.zeros((npad,), F32)])

    deg0, deg1 = _make_deg_kernel(npad, kg)(dst, init)
    d0 = deg0.reshape(npad, 1)
    d1 = deg1.reshape(npad, 1)

    dspec = _row_spec(rb, 1)
    agg = _make_agg_kernel(npad, h, kg)

    xs0, xs1 = pl.pallas_call(
        _prescale_body,
        grid=(grid,),
        in_specs=[_row_spec(rb, din), dspec, dspec, _full_spec((din, dh))],
        out_specs=[_row_spec(rb, h), _row_spec(rb, h)],
        out_shape=[jax.ShapeDtypeStruct((npad, h), F32)] * 2,
    )(xp, d0, d1, W0)

    for b, w in ((b0, W1), (b1, W2)):
        a0, a1 = agg(xs0, xs1, src, dst)
        xs0, xs1 = pl.pallas_call(
            _layer_body,
            grid=(grid,),
            in_specs=[_row_spec(rb, h), _row_spec(rb, h), dspec, dspec,
                      _full_spec((1, dh)), _full_spec((dh, dh))],
            out_specs=[_row_spec(rb, h), _row_spec(rb, h)],
            out_shape=[jax.ShapeDtypeStruct((npad, h), F32)] * 2,
        )(a0, a1, d0, d1, b.reshape(1, dh), w)

    a0, a1 = agg(xs0, xs1, src, dst)
    out = pl.pallas_call(
        _final_body,
        grid=(grid,),
        in_specs=[_row_spec(rb, h), _row_spec(rb, h), dspec, dspec,
                  _full_spec((1, dh)), _full_spec((dh, dout)),
                  _full_spec((1, dout))],
        out_specs=_row_spec(rb, dout),
        out_shape=jax.ShapeDtypeStruct((npad, dout), F32),
    )(a0, a1, d0, d1, b2.reshape(1, dh), Wl, bl.reshape(1, dout))
    return out[:n]


# speedup vs baseline: 9.3227x; 1.0006x over previous
import functools

import jax
import jax.numpy as jnp
from jax import lax
from jax.experimental import pallas as pl
from jax.experimental.pallas import tpu as pltpu
from jax.experimental.pallas import tpu_sc as plsc

F32 = jnp.float32
NTILES = 16
CH = 128
GRP = 16


def _prescale_body(x_ref, d0_ref, d1_ref, w_ref, o0_ref, o1_ref):
    h = o0_ref.shape[1]
    dinv = lax.rsqrt(d0_ref[...] + d1_ref[...])
    xw = jnp.dot(x_ref[...], w_ref[...], preferred_element_type=F32)
    xs = xw * dinv
    o0_ref[...] = xs[:, :h]
    o1_ref[...] = xs[:, h:]


def _layer_body(a0_ref, a1_ref, d0_ref, d1_ref, b_ref, w_ref, o0_ref, o1_ref):
    h = o0_ref.shape[1]
    dinv = lax.rsqrt(d0_ref[...] + d1_ref[...])
    agg = jnp.concatenate([a0_ref[...], a1_ref[...]], axis=1)
    act = jnp.maximum(agg * dinv + b_ref[...], 0.0)
    xw = jnp.dot(act, w_ref[...], preferred_element_type=F32)
    xs = xw * dinv
    o0_ref[...] = xs[:, :h]
    o1_ref[...] = xs[:, h:]


def _final_body(a0_ref, a1_ref, d0_ref, d1_ref, b_ref, w_ref, bl_ref, o_ref):
    dinv = lax.rsqrt(d0_ref[...] + d1_ref[...])
    agg = jnp.concatenate([a0_ref[...], a1_ref[...]], axis=1)
    act = jnp.maximum(agg * dinv + b_ref[...], 0.0)
    o_ref[...] = jnp.dot(act, w_ref[...], preferred_element_type=F32) + bl_ref[...]


def _make_deg_kernel(npad, kg):
    rpt = npad // NTILES
    kg2 = kg // 2
    mesh = plsc.VectorSubcoreMesh(core_axis_name="c", subcore_axis_name="s",
                                  num_cores=2, num_subcores=NTILES)

    @functools.partial(
        pl.kernel,
        out_type=(jax.ShapeDtypeStruct((npad,), F32),
                  jax.ShapeDtypeStruct((npad,), F32)),
        mesh=mesh,
        scratch_types=[
            pltpu.VMEM_SHARED((npad,), F32),
            pltpu.VMEM((GRP, CH), jnp.int32),
            pltpu.VMEM((CH,), F32),
        ],
    )
    def deg_kernel(dst_hbm, init_hbm, out0, out1, dacc, dst_b, ones_v):
        c = lax.axis_index("c")
        s = lax.axis_index("s")
        r0 = s * rpt
        pltpu.sync_copy(init_hbm.at[pl.ds(c * npad + r0, rpt)],
                        dacc.at[pl.ds(r0, rpt)])
        pltpu.sync_copy(init_hbm.at[pl.ds(0, CH)], ones_v)
        plsc.subcore_barrier()

        @pl.loop(c * kg2, (c + 1) * kg2)
        def _(g):
            pltpu.sync_copy(dst_hbm.at[s, g], dst_b)
            for j in range(GRP):
                pltpu.sync_copy(ones_v, dacc.at[dst_b.at[j]], add=True)

        plsc.subcore_barrier()

        @pl.when(c == 0)
        def _():
            pltpu.sync_copy(dacc.at[pl.ds(r0, rpt)], out0.at[pl.ds(r0, rpt)])

        @pl.when(c == 1)
        def _():
            pltpu.sync_copy(dacc.at[pl.ds(r0, rpt)], out1.at[pl.ds(r0, rpt)])

    return deg_kernel


def _make_agg_kernel(npad, h, kg):
    rpt = npad // NTILES
    mesh = plsc.VectorSubcoreMesh(core_axis_name="c", subcore_axis_name="s",
                                  num_cores=2, num_subcores=NTILES)

    @functools.partial(
        pl.kernel,
        out_type=(jax.ShapeDtypeStruct((npad, h), F32),
                  jax.ShapeDtypeStruct((npad, h), F32)),
        mesh=mesh,
        scratch_types=[
            pltpu.VMEM_SHARED((npad, h), F32),
            pltpu.VMEM((GRP, CH), jnp.int32),
            pltpu.VMEM((GRP, CH), jnp.int32),
            pltpu.VMEM((CH, h), F32),
            pltpu.VMEM((CH, h), F32),
            pltpu.SemaphoreType.DMA,
            pltpu.SemaphoreType.DMA,
            pltpu.SemaphoreType.DMA,
            pltpu.SemaphoreType.DMA,
        ],
    )
    def agg_kernel(xs0, xs1, src_hbm, dst_hbm, out0, out1,
                   acc, src_b, dst_b, rows_a, rows_b,
                   gsem_a, gsem_b, ssem_a, ssem_b):
        c = lax.axis_index("c")
        s = lax.axis_index("s")
        r0 = s * rpt
        bufs = ((rows_a, gsem_a, ssem_a), (rows_b, gsem_b, ssem_b))

        def run(xs_c, out_c):
            pltpu.sync_copy(xs_c.at[pl.ds(r0, rpt)], acc.at[pl.ds(r0, rpt)])
            plsc.subcore_barrier()

            @pl.loop(0, kg)
            def _(g):
                pltpu.sync_copy(src_hbm.at[s, g], src_b)
                pltpu.sync_copy(dst_hbm.at[s, g], dst_b)
                pltpu.async_copy(xs_c.at[src_b.at[0]], rows_a, gsem_a)
                for j in range(GRP):
                    rows, gsem, ssem = bufs[j % 2]
                    nrows, ngsem, nssem = bufs[(j + 1) % 2]
                    if j + 1 < GRP:
                        if j >= 1:
                            pltpu.make_async_copy(
                                nrows, acc.at[dst_b.at[j - 1]], nssem).wait()
                        pltpu.async_copy(xs_c.at[src_b.at[j + 1]], nrows, ngsem)
                    pltpu.make_async_copy(xs_c.at[src_b.at[j]], rows, gsem).wait()
                    pltpu.async_copy(rows, acc.at[dst_b.at[j]], ssem, add=True)
                for j in (GRP - 2, GRP - 1):
                    rows, _, ssem = bufs[j % 2]
                    pltpu.make_async_copy(rows, acc.at[dst_b.at[j]], ssem).wait()

            plsc.subcore_barrier()
            pltpu.sync_copy(acc.at[pl.ds(r0, rpt)], out_c.at[pl.ds(r0, rpt)])

        @pl.when(c == 0)
        def _():
            run(xs0, out0)

        @pl.when(c == 1)
        def _():
            run(xs1, out1)

    return agg_kernel


def _row_spec(rb, w):
    return pl.BlockSpec((rb, w), lambda i: (i, 0))


def _full_spec(shape):
    nd = len(shape)
    return pl.BlockSpec(shape, lambda i: (0,) * nd)


@jax.jit
def kernel(x, edge_index, W0, b0, W1, b1, W2, b2, Wl, bl):
    n, din = x.shape
    e = edge_index.shape[1]
    dh = W0.shape[1]
    dout = Wl.shape[1]
    h = dh // 2

    rb = 1024
    npad = pl.cdiv(n + NTILES, rb) * rb
    epb = NTILES * CH * GRP * 2
    epad = pl.cdiv(e, epb) * epb
    kg = epad // (NTILES * CH * GRP)
    grid = npad // rb

    pad_e = epad - e
    src = jnp.concatenate(
        [edge_index[0],
         jnp.zeros((pad_e,), jnp.int32)]).reshape(NTILES, kg, GRP, CH)
    dst = jnp.concatenate(
        [edge_index[1],
         n + (jnp.arange(pad_e, dtype=jnp.int32) % NTILES)]
    ).reshape(NTILES, kg, GRP, CH)
    xp = jnp.zeros((npad, din), F32).at[:n].set(x)
    init = jnp.concatenate([jnp.ones((npad,), F32), jnp.zeros((npad,), F32)])

    deg0, deg1 = _make_deg_kernel(npad, kg)(dst, init)
    d0 = deg0.reshape(npad, 1)
    d1 = deg1.reshape(npad, 1)

    dspec = _row_spec(rb, 1)
    agg = _make_agg_kernel(npad, h, kg)

    xs0, xs1 = pl.pallas_call(
        _prescale_body,
        grid=(grid,),
        in_specs=[_row_spec(rb, din), dspec, dspec, _full_spec((din, dh))],
        out_specs=[_row_spec(rb, h), _row_spec(rb, h)],
        out_shape=[jax.ShapeDtypeStruct((npad, h), F32)] * 2,
    )(xp, d0, d1, W0)

    for b, w in ((b0, W1), (b1, W2)):
        a0, a1 = agg(xs0, xs1, src, dst)
        xs0, xs1 = pl.pallas_call(
            _layer_body,
            grid=(grid,),
            in_specs=[_row_spec(rb, h), _row_spec(rb, h), dspec, dspec,
                      _full_spec((1, dh)), _full_spec((dh, dh))],
            out_specs=[_row_spec(rb, h), _row_spec(rb, h)],
            out_shape=[jax.ShapeDtypeStruct((npad, h), F32)] * 2,
        )(a0, a1, d0, d1, b.reshape(1, dh), w)

    a0, a1 = agg(xs0, xs1, src, dst)
    out = pl.pallas_call(
        _final_body,
        grid=(grid,),
        in_specs=[_row_spec(rb, h), _row_spec(rb, h), dspec, dspec,
                  _full_spec((1, dh)), _full_spec((dh, dout)),
                  _full_spec((1, dout))],
        out_specs=_row_spec(rb, dout),
        out_shape=jax.ShapeDtypeStruct((npad, dout), F32),
    )(a0, a1, d0, d1, b2.reshape(1, dh), Wl, bl.reshape(1, dout))
    return out[:n]
